# baseline (speedup 1.0000x reference)
import functools

import jax
import jax.numpy as jnp
from jax import lax
from jax.experimental import pallas as pl
from jax.experimental.pallas import tpu as pltpu
from jax.experimental.pallas import tpu_sc as plsc

T = 2048
D = 2048
H = 16
KV = 4
DH = 128
E = 8
TOPK = 2
I = 768
EPS = 1e-6
BASE = 1000000.0

BT = 256
BQ = 256
BK = 256
BR = 256
BM = 128
NBCAP = (TOPK * T) // BM + E
CAP = NBCAP * BM

_F32 = jnp.float32


def _rms_qkv_body(x_ref, w_ref, wqkv_ref, o_ref):
    x = x_ref[...]
    var = jnp.mean(x * x, axis=-1, keepdims=True)
    xn = x * lax.rsqrt(var + EPS) * w_ref[...]
    o_ref[...] = jnp.dot(xn, wqkv_ref[...], preferred_element_type=_F32,
                         precision=lax.Precision.HIGHEST)


def _rms_qkv(x, w, wqkv):
    return pl.pallas_call(
        _rms_qkv_body,
        grid=(T // BT,),
        in_specs=[
            pl.BlockSpec((BT, D), lambda i: (i, 0)),
            pl.BlockSpec((1, D), lambda i: (0, 0)),
            pl.BlockSpec((D, (H + 2 * KV) * DH), lambda i: (0, 0)),
        ],
        out_specs=pl.BlockSpec((BT, (H + 2 * KV) * DH), lambda i: (i, 0)),
        out_shape=jax.ShapeDtypeStruct((T, (H + 2 * KV) * DH), _F32),
    )(x, w, wqkv)


def _rope_body(x_ref, nw_ref, cos_ref, sin_ref, o_ref):
    x = x_ref[...]
    var = jnp.mean(x * x, axis=-1, keepdims=True)
    xn = x * lax.rsqrt(var + EPS) * nw_ref[0]
    half = DH // 2
    x1 = xn[:, :half]
    x2 = xn[:, half:]
    c = cos_ref[...]
    s = sin_ref[...]
    o_ref[...] = jnp.concatenate([x1 * c - x2 * s, x2 * c + x1 * s], axis=-1)


def _rope_qk(qkv, qkn, cos, sin):
    return pl.pallas_call(
        _rope_body,
        grid=(T // BT, H + KV),
        in_specs=[
            pl.BlockSpec((BT, DH), lambda i, j: (i, j)),
            pl.BlockSpec((1, 1, DH), lambda i, j: (j // H, 0, 0)),
            pl.BlockSpec((BT, DH // 2), lambda i, j: (i, 0)),
            pl.BlockSpec((BT, DH // 2), lambda i, j: (i, 0)),
        ],
        out_specs=pl.BlockSpec((BT, DH), lambda i, j: (i, j)),
        out_shape=jax.ShapeDtypeStruct((T, (H + KV) * DH), _F32),
    )(qkv, qkn, cos, sin)


def _flash_body(q_ref, k_ref, v_ref, o_ref):
    qi = pl.program_id(1)
    scale = DH ** -0.5
    q = q_ref[...] * scale

    def body(kb, carry):
        m, l, acc = carry
        k = k_ref[pl.ds(kb * BK, BK), :]
        s = lax.dot_general(q, k, (((1,), (1,)), ((), ())),
                            preferred_element_type=_F32,
                            precision=lax.Precision.HIGHEST)
        rowi = qi * BQ + lax.broadcasted_iota(jnp.int32, (BQ, BK), 0)
        coli = kb * BK + lax.broadcasted_iota(jnp.int32, (BQ, BK), 1)
        s = jnp.where(coli <= rowi, s, -1e30)
        m_new = jnp.maximum(m, jnp.max(s, axis=-1, keepdims=True))
        p = jnp.exp(s - m_new)
        alpha = jnp.exp(m - m_new)
        l_new = l * alpha + jnp.sum(p, axis=-1, keepdims=True)
        pv = jnp.dot(p, v_ref[pl.ds(kb * BK, BK), :],
                     preferred_element_type=_F32,
                     precision=lax.Precision.HIGHEST)
        return m_new, l_new, acc * alpha + pv

    m0 = jnp.full((BQ, 1), -1e30, _F32)
    l0 = jnp.zeros((BQ, 1), _F32)
    a0 = jnp.zeros((BQ, DH), _F32)
    m, l, acc = lax.fori_loop(0, qi + 1, body, (m0, l0, a0))
    o_ref[...] = acc / l


def _flash(qk, qkv):
    rep = H // KV
    return pl.pallas_call(
        _flash_body,
        grid=(H, T // BQ),
        in_specs=[
            pl.BlockSpec((BQ, DH), lambda h, qi: (qi, h)),
            pl.BlockSpec((T, DH), lambda h, qi: (0, H + h // rep)),
            pl.BlockSpec((T, DH), lambda h, qi: (0, H + KV + h // rep)),
        ],
        out_specs=pl.BlockSpec((BQ, DH), lambda h, qi: (qi, h)),
        out_shape=jax.ShapeDtypeStruct((T, H * DH), _F32),
    )(qk, qk, qkv)


def _oproj_body(a_ref, wo_ref, r_ref, pw_ref, wg_ref, nr_ref, h2_ref, lg_ref):
    acc = jnp.dot(a_ref[...], wo_ref[...], preferred_element_type=_F32,
                  precision=lax.Precision.HIGHEST)
    acc = acc + r_ref[...]
    nr_ref[...] = acc
    var = jnp.mean(acc * acc, axis=-1, keepdims=True)
    h2 = acc * lax.rsqrt(var + EPS) * pw_ref[...]
    h2_ref[...] = h2
    lg_ref[...] = jnp.dot(h2, wg_ref[...], preferred_element_type=_F32,
                          precision=lax.Precision.HIGHEST)


def _oproj(attn, wo, resid, pw, wgate):
    return pl.pallas_call(
        _oproj_body,
        grid=(T // BT,),
        in_specs=[
            pl.BlockSpec((BT, H * DH), lambda i: (i, 0)),
            pl.BlockSpec((H * DH, D), lambda i: (0, 0)),
            pl.BlockSpec((BT, D), lambda i: (i, 0)),
            pl.BlockSpec((1, D), lambda i: (0, 0)),
            pl.BlockSpec((D, E), lambda i: (0, 0)),
        ],
        out_specs=[
            pl.BlockSpec((BT, D), lambda i: (i, 0)),
            pl.BlockSpec((BT, D), lambda i: (i, 0)),
            pl.BlockSpec((BT, E), lambda i: (i, 0)),
        ],
        out_shape=[
            jax.ShapeDtypeStruct((T, D), _F32),
            jax.ShapeDtypeStruct((T, D), _F32),
            jax.ShapeDtypeStruct((T, E), _F32),
        ],
    )(attn, wo, resid, pw, wgate)


def _route_body(lg_ref, idx_ref, wts_ref, cum_ref, cnt_ref, carry):
    i = pl.program_id(0)

    @pl.when(i == 0)
    def _():
        carry[...] = jnp.zeros((1, E), _F32)

    lg = lg_ref[...]
    mx = jnp.max(lg, axis=-1, keepdims=True)
    ex = jnp.exp(lg - mx)
    p = ex / jnp.sum(ex, axis=-1, keepdims=True)
    lane = lax.broadcasted_iota(jnp.int32, (BR, E), 1)
    v0 = jnp.max(p, axis=-1, keepdims=True)
    i0 = jnp.min(jnp.where(p == v0, lane, E), axis=-1, keepdims=True)
    p1 = jnp.where(lane == i0, -1.0, p)
    v1 = jnp.max(p1, axis=-1, keepdims=True)
    i1 = jnp.min(jnp.where(p1 == v1, lane, E), axis=-1, keepdims=True)
    sw = v0 + v1
    idx_ref[...] = jnp.concatenate([i0, i1], axis=-1)
    wts_ref[...] = jnp.concatenate([v0 / sw, v1 / sw], axis=-1)

    m = (lane == i0).astype(_F32) + (lane == i1).astype(_F32)
    r = lax.broadcasted_iota(jnp.int32, (BR, BR), 0)
    c = lax.broadcasted_iota(jnp.int32, (BR, BR), 1)
    tril = (c < r).astype(_F32)
    cum_ref[...] = jnp.dot(tril, m, preferred_element_type=_F32) + carry[...]
    carry[...] = carry[...] + jnp.sum(m, axis=0, keepdims=True)
    cnt_ref[...] = carry[...]


def _route(logits):
    return pl.pallas_call(
        _route_body,
        grid=(T // BR,),
        in_specs=[pl.BlockSpec((BR, E), lambda i: (i, 0))],
        out_specs=[
            pl.BlockSpec((BR, TOPK), lambda i: (i, 0)),
            pl.BlockSpec((BR, TOPK), lambda i: (i, 0)),
            pl.BlockSpec((BR, E), lambda i: (i, 0)),
            pl.BlockSpec((1, E), lambda i: (0, 0)),
        ],
        out_shape=[
            jax.ShapeDtypeStruct((T, TOPK), jnp.int32),
            jax.ShapeDtypeStruct((T, TOPK), _F32),
            jax.ShapeDtypeStruct((T, E), _F32),
            jax.ShapeDtypeStruct((1, E), _F32),
        ],
        scratch_shapes=[pltpu.VMEM((1, E), _F32)],
    )(logits)


def _meta_body(cnt_ref, poff_ref, eid_ref, nreal_ref):
    cnt = cnt_ref[...]
    pc = jnp.ceil(cnt / BM) * BM
    ei = lax.broadcasted_iota(jnp.int32, (E, E), 0)
    ej = lax.broadcasted_iota(jnp.int32, (E, E), 1)
    tri = (ei < ej).astype(_F32)
    poff = jnp.dot(pc, tri, preferred_element_type=_F32)
    poff_ref[...] = poff
    bnd = (poff + pc) / BM
    gi = lax.broadcasted_iota(jnp.int32, (NBCAP, E), 0).astype(_F32)
    eid = jnp.sum((gi >= bnd).astype(_F32), axis=-1, keepdims=True)
    eid_ref[...] = jnp.minimum(eid, E - 1).astype(jnp.int32)
    nreal_ref[...] = (jnp.sum(pc, axis=-1, keepdims=True) / BM).astype(jnp.int32)


def _meta(cnt):
    return pl.pallas_call(
        _meta_body,
        grid=(1,),
        in_specs=[pl.BlockSpec((1, E), lambda i: (0, 0))],
        out_specs=[
            pl.BlockSpec((1, E), lambda i: (0, 0)),
            pl.BlockSpec((NBCAP, 1), lambda i: (0, 0)),
            pl.BlockSpec((1, 1), lambda i: (0, 0)),
        ],
        out_shape=[
            jax.ShapeDtypeStruct((1, E), _F32),
            jax.ShapeDtypeStruct((NBCAP, 1), jnp.int32),
            jax.ShapeDtypeStruct((1, 1), jnp.int32),
        ],
    )(cnt)


def _pos_body(idx_ref, cum_ref, poff_ref, pos_ref):
    lane = lax.broadcasted_iota(jnp.int32, (BR, E), 1)
    base = cum_ref[...] + poff_ref[...]
    i0 = idx_ref[:, 0:1]
    i1 = idx_ref[:, 1:2]
    p0 = jnp.sum(jnp.where(lane == i0, base, 0.0), axis=-1, keepdims=True)
    p1 = jnp.sum(jnp.where(lane == i1, base, 0.0), axis=-1, keepdims=True)
    pos_ref[...] = jnp.concatenate([p0, p1], axis=-1).astype(jnp.int32)


def _pos(idx, cum, poff):
    return pl.pallas_call(
        _pos_body,
        grid=(T // BR,),
        in_specs=[
            pl.BlockSpec((BR, TOPK), lambda i: (i, 0)),
            pl.BlockSpec((BR, E), lambda i: (i, 0)),
            pl.BlockSpec((1, E), lambda i: (0, 0)),
        ],
        out_specs=pl.BlockSpec((BR, TOPK), lambda i: (i, 0)),
        out_shape=jax.ShapeDtypeStruct((T, TOPK), jnp.int32),
    )(idx, cum, poff)


def _gmm_body(eid_ref, nreal_ref, hg_ref, wg_ref, wu_ref, wd_ref, o_ref):
    g = pl.program_id(0)

    @pl.when(g < nreal_ref[0, 0])
    def _():
        a = hg_ref[...]
        gg = jnp.dot(a, wg_ref[0], preferred_element_type=_F32,
                     precision=lax.Precision.HIGHEST)
        uu = jnp.dot(a, wu_ref[0], preferred_element_type=_F32,
                     precision=lax.Precision.HIGHEST)
        act = gg * jax.nn.sigmoid(gg) * uu
        o_ref[...] = jnp.dot(act, wd_ref[0], preferred_element_type=_F32,
                             precision=lax.Precision.HIGHEST)


def _gmm(eid, nreal, hg, wg, wu, wd):
    grid_spec = pltpu.PrefetchScalarGridSpec(
        num_scalar_prefetch=2,
        grid=(NBCAP,),
        in_specs=[
            pl.BlockSpec((BM, D), lambda g, eid, nr: (g, 0)),
            pl.BlockSpec((1, D, I), lambda g, eid, nr: (eid[g, 0], 0, 0)),
            pl.BlockSpec((1, D, I), lambda g, eid, nr: (eid[g, 0], 0, 0)),
            pl.BlockSpec((1, I, D), lambda g, eid, nr: (eid[g, 0], 0, 0)),
        ],
        out_specs=pl.BlockSpec((BM, D), lambda g, eid, nr: (g, 0)),
    )
    return pl.pallas_call(
        _gmm_body,
        grid_spec=grid_spec,
        out_shape=jax.ShapeDtypeStruct((CAP, D), _F32),
    )(eid, nreal, hg, wg, wu, wd)


_NW = 32
_PW = (TOPK * T) // _NW
_NCH = _PW // 16
_TW = T // _NW


def _sc_mesh():
    return plsc.VectorSubcoreMesh(core_axis_name="c", subcore_axis_name="s")


def _moe_dispatch(h2, pos3, tok3):
    @functools.partial(
        pl.kernel,
        out_type=jax.ShapeDtypeStruct((CAP, D), _F32),
        mesh=_sc_mesh(),
        scratch_types=[
            pltpu.VMEM((_NCH, 16), jnp.int32),
            pltpu.VMEM((_NCH, 16), jnp.int32),
            pltpu.VMEM((16, D), _F32),
            pltpu.VMEM((16, D), _F32),
            pltpu.SemaphoreType.DMA,
            pltpu.SemaphoreType.DMA,
            pltpu.SemaphoreType.DMA,
            pltpu.SemaphoreType.DMA,
        ],
    )
    def k(h2_hbm, pos_hbm, tok_hbm, out_hbm, tokv, posv, bufa, bufb,
          ga, gb, sa, sb):
        wid = lax.axis_index("s") * 2 + lax.axis_index("c")
        pltpu.sync_copy(tok_hbm.at[wid], tokv)
        pltpu.sync_copy(pos_hbm.at[wid], posv)
        bufs = (bufa, bufb)
        gsem = (ga, gb)
        ssem = (sa, sb)
        gathers = [None] * _NCH
        scat = [None, None]
        gathers[0] = pltpu.async_copy(h2_hbm.at[tokv.at[0]], bufs[0], gsem[0])
        for j in range(_NCH):
            b = j % 2
            gathers[j].wait()
            if j + 1 < _NCH:
                nb = (j + 1) % 2
                if scat[nb] is not None:
                    scat[nb].wait()
                gathers[j + 1] = pltpu.async_copy(
                    h2_hbm.at[tokv.at[j + 1]], bufs[nb], gsem[nb])
            scat[b] = pltpu.async_copy(bufs[b], out_hbm.at[posv.at[j]],
                                       ssem[b])
        for b in range(2):
            if scat[b] is not None:
                scat[b].wait()

    return k(h2, pos3, tok3)


def _moe_combine(ys, pos3, wts3):
    @functools.partial(
        pl.kernel,
        out_type=jax.ShapeDtypeStruct((T, D), _F32),
        mesh=_sc_mesh(),
        scratch_types=[
            pltpu.VMEM((_NCH, 16), jnp.int32),
            pltpu.VMEM((_NCH, 16), _F32),
            pltpu.VMEM((16, D), _F32),
            pltpu.VMEM((8, D), _F32),
            pltpu.SemaphoreType.DMA,
            pltpu.SemaphoreType.DMA,
        ],
    )
    def k(ys_hbm, pos_hbm, wts_hbm, out_hbm, posv, wv, ybuf, obuf, gs, ss):
        wid = lax.axis_index("s") * 2 + lax.axis_index("c")
        pltpu.sync_copy(pos_hbm.at[wid], posv)
        pltpu.sync_copy(wts_hbm.at[wid], wv)
        base = wid * _TW
        for j in range(_NCH):
            pltpu.async_copy(ys_hbm.at[posv.at[j]], ybuf, gs).wait()
            wrow = wv[j]
            for i in range(8):
                w0 = wrow[2 * i]
                w1 = wrow[2 * i + 1]

                def body(c, carry, i=i, w0=w0, w1=w1):
                    sl = pl.ds(c * 16, 16)
                    obuf[i, sl] = w0 * ybuf[2 * i, sl] + w1 * ybuf[2 * i + 1, sl]
                    return carry

                lax.fori_loop(0, D // 16, body, 0)
            pltpu.sync_copy(obuf, out_hbm.at[pl.ds(base + 8 * j, 8)])

    return k(ys, pos3, wts3)


def kernel(positions, hidden_states, in_ln_w, post_ln_w, q_norm_w, k_norm_w,
           Wq, Wk, Wv, Wo, Wgate, Wg, Wu, Wd):
    wqkv = jnp.concatenate([Wq, Wk, Wv], axis=1)
    half = DH // 2
    inv_freq = 1.0 / (BASE ** (jnp.arange(0, half, dtype=_F32) / half))
    freqs = positions.astype(_F32)[:, None] * inv_freq[None, :]
    cos = jnp.cos(freqs)
    sin = jnp.sin(freqs)
    qkn = jnp.stack([q_norm_w, k_norm_w], axis=0).reshape(2, 1, DH)

    qkv = _rms_qkv(hidden_states, in_ln_w.reshape(1, D), wqkv)
    qk = _rope_qk(qkv[:, : (H + KV) * DH], qkn, cos, sin)
    attn = _flash(qk, qkv)
    new_res, h2, logits = _oproj(attn, Wo, hidden_states,
                                 post_ln_w.reshape(1, D), Wgate)
    idx, wts, cum, cnt = _route(logits)
    poff, eid, nreal = _meta(cnt)
    pos = _pos(idx, cum, poff)

    tok3 = jax.lax.broadcasted_iota(
        jnp.int32, (T, TOPK), 0).reshape(_NW, _NCH, 16)
    pos3 = pos.reshape(_NW, _NCH, 16)
    wts3 = wts.reshape(_NW, _NCH, 16)
    hg = _moe_dispatch(h2, pos3, tok3)
    ys = _gmm(eid, nreal, hg, Wg, Wu, Wd)
    moe_out = _moe_combine(ys, pos3, wts3)
    return (moe_out, new_res)


# speedup vs baseline: 2.2661x; 2.2661x over previous
import functools

import jax
import jax.numpy as jnp
from jax import lax
from jax.experimental import pallas as pl
from jax.experimental.pallas import tpu as pltpu
from jax.experimental.pallas import tpu_sc as plsc

T = 2048
D = 2048
H = 16
KV = 4
DH = 128
E = 8
TOPK = 2
I = 768
EPS = 1e-6
BASE = 1000000.0

BT = 256
BQ = 256
BK = 256
BR = 256
BM = 128
NBCAP = (TOPK * T) // BM + E
CAP = NBCAP * BM

_F32 = jnp.float32


def _rms_qkv_body(x_ref, w_ref, wqkv_ref, o_ref):
    x = x_ref[...]
    var = jnp.mean(x * x, axis=-1, keepdims=True)
    xn = x * lax.rsqrt(var + EPS) * w_ref[...]
    o_ref[...] = jnp.dot(xn, wqkv_ref[...], preferred_element_type=_F32)


def _rms_qkv(x, w, wqkv):
    return pl.pallas_call(
        _rms_qkv_body,
        grid=(T // BT,),
        in_specs=[
            pl.BlockSpec((BT, D), lambda i: (i, 0)),
            pl.BlockSpec((1, D), lambda i: (0, 0)),
            pl.BlockSpec((D, (H + 2 * KV) * DH), lambda i: (0, 0)),
        ],
        out_specs=pl.BlockSpec((BT, (H + 2 * KV) * DH), lambda i: (i, 0)),
        out_shape=jax.ShapeDtypeStruct((T, (H + 2 * KV) * DH), _F32),
    )(x, w, wqkv)


def _rope_body(x_ref, nw_ref, cos_ref, sin_ref, o_ref):
    x = x_ref[...]
    var = jnp.mean(x * x, axis=-1, keepdims=True)
    xn = x * lax.rsqrt(var + EPS) * nw_ref[0]
    half = DH // 2
    x1 = xn[:, :half]
    x2 = xn[:, half:]
    c = cos_ref[...]
    s = sin_ref[...]
    o_ref[...] = jnp.concatenate([x1 * c - x2 * s, x2 * c + x1 * s], axis=-1)


def _rope_qk(qkv, qkn, cos, sin):
    return pl.pallas_call(
        _rope_body,
        grid=(T // BT, H + KV),
        in_specs=[
            pl.BlockSpec((BT, DH), lambda i, j: (i, j)),
            pl.BlockSpec((1, 1, DH), lambda i, j: (j // H, 0, 0)),
            pl.BlockSpec((BT, DH // 2), lambda i, j: (i, 0)),
            pl.BlockSpec((BT, DH // 2), lambda i, j: (i, 0)),
        ],
        out_specs=pl.BlockSpec((BT, DH), lambda i, j: (i, j)),
        out_shape=jax.ShapeDtypeStruct((T, (H + KV) * DH), _F32),
    )(qkv, qkn, cos, sin)


def _flash_body(q_ref, k_ref, v_ref, o_ref):
    qi = pl.program_id(1)
    scale = DH ** -0.5
    s = lax.dot_general(q_ref[...], k_ref[...], (((1,), (1,)), ((), ())),
                        preferred_element_type=_F32)
    s = s * scale
    rowi = qi * BQ + lax.broadcasted_iota(jnp.int32, (BQ, T), 0)
    coli = lax.broadcasted_iota(jnp.int32, (BQ, T), 1)
    s = jnp.where(coli <= rowi, s, jnp.finfo(_F32).min)
    mx = jnp.max(s, axis=-1, keepdims=True)
    p = jnp.exp(s - mx)
    l = jnp.sum(p, axis=-1, keepdims=True)
    p = p / l
    o_ref[...] = jnp.dot(p, v_ref[...], preferred_element_type=_F32)


def _flash(qk, qkv):
    rep = H // KV
    return pl.pallas_call(
        _flash_body,
        grid=(H, T // BQ),
        in_specs=[
            pl.BlockSpec((BQ, DH), lambda h, qi: (qi, h)),
            pl.BlockSpec((T, DH), lambda h, qi: (0, H + h // rep)),
            pl.BlockSpec((T, DH), lambda h, qi: (0, H + KV + h // rep)),
        ],
        out_specs=pl.BlockSpec((BQ, DH), lambda h, qi: (qi, h)),
        out_shape=jax.ShapeDtypeStruct((T, H * DH), _F32),
    )(qk, qk, qkv)


def _oproj_body(a_ref, wo_ref, r_ref, pw_ref, wg_ref, nr_ref, h2_ref, lg_ref):
    acc = jnp.dot(a_ref[...], wo_ref[...], preferred_element_type=_F32)
    acc = acc + r_ref[...]
    nr_ref[...] = acc
    var = jnp.mean(acc * acc, axis=-1, keepdims=True)
    h2 = acc * lax.rsqrt(var + EPS) * pw_ref[...]
    h2_ref[...] = h2
    lg_ref[...] = jnp.dot(h2, wg_ref[...], preferred_element_type=_F32)


def _oproj(attn, wo, resid, pw, wgate):
    return pl.pallas_call(
        _oproj_body,
        grid=(T // BT,),
        in_specs=[
            pl.BlockSpec((BT, H * DH), lambda i: (i, 0)),
            pl.BlockSpec((H * DH, D), lambda i: (0, 0)),
            pl.BlockSpec((BT, D), lambda i: (i, 0)),
            pl.BlockSpec((1, D), lambda i: (0, 0)),
            pl.BlockSpec((D, E), lambda i: (0, 0)),
        ],
        out_specs=[
            pl.BlockSpec((BT, D), lambda i: (i, 0)),
            pl.BlockSpec((BT, D), lambda i: (i, 0)),
            pl.BlockSpec((BT, E), lambda i: (i, 0)),
        ],
        out_shape=[
            jax.ShapeDtypeStruct((T, D), _F32),
            jax.ShapeDtypeStruct((T, D), _F32),
            jax.ShapeDtypeStruct((T, E), _F32),
        ],
    )(attn, wo, resid, pw, wgate)


def _route_body(lg_ref, idx_ref, wts_ref, cum_ref, cnt_ref, carry):
    i = pl.program_id(0)

    @pl.when(i == 0)
    def _():
        carry[...] = jnp.zeros((1, E), _F32)

    lg = lg_ref[...]
    mx = jnp.max(lg, axis=-1, keepdims=True)
    ex = jnp.exp(lg - mx)
    p = ex / jnp.sum(ex, axis=-1, keepdims=True)
    lane = lax.broadcasted_iota(jnp.int32, (BR, E), 1)
    v0 = jnp.max(p, axis=-1, keepdims=True)
    i0 = jnp.min(jnp.where(p == v0, lane, E), axis=-1, keepdims=True)
    p1 = jnp.where(lane == i0, -1.0, p)
    v1 = jnp.max(p1, axis=-1, keepdims=True)
    i1 = jnp.min(jnp.where(p1 == v1, lane, E), axis=-1, keepdims=True)
    sw = v0 + v1
    idx_ref[...] = jnp.concatenate([i0, i1], axis=-1)
    wts_ref[...] = jnp.concatenate([v0 / sw, v1 / sw], axis=-1)

    m = (lane == i0).astype(_F32) + (lane == i1).astype(_F32)
    r = lax.broadcasted_iota(jnp.int32, (BR, BR), 0)
    c = lax.broadcasted_iota(jnp.int32, (BR, BR), 1)
    tril = (c < r).astype(_F32)
    cum_ref[...] = jnp.dot(tril, m, preferred_element_type=_F32) + carry[...]
    carry[...] = carry[...] + jnp.sum(m, axis=0, keepdims=True)
    cnt_ref[...] = carry[...]


def _route(logits):
    return pl.pallas_call(
        _route_body,
        grid=(T // BR,),
        in_specs=[pl.BlockSpec((BR, E), lambda i: (i, 0))],
        out_specs=[
            pl.BlockSpec((BR, TOPK), lambda i: (i, 0)),
            pl.BlockSpec((BR, TOPK), lambda i: (i, 0)),
            pl.BlockSpec((BR, E), lambda i: (i, 0)),
            pl.BlockSpec((1, E), lambda i: (0, 0)),
        ],
        out_shape=[
            jax.ShapeDtypeStruct((T, TOPK), jnp.int32),
            jax.ShapeDtypeStruct((T, TOPK), _F32),
            jax.ShapeDtypeStruct((T, E), _F32),
            jax.ShapeDtypeStruct((1, E), _F32),
        ],
        scratch_shapes=[pltpu.VMEM((1, E), _F32)],
    )(logits)


def _meta_body(cnt_ref, poff_ref, eid_ref, nreal_ref):
    cnt = cnt_ref[...]
    pc = jnp.ceil(cnt / BM) * BM
    ei = lax.broadcasted_iota(jnp.int32, (E, E), 0)
    ej = lax.broadcasted_iota(jnp.int32, (E, E), 1)
    tri = (ei < ej).astype(_F32)
    poff = jnp.dot(pc, tri, preferred_element_type=_F32)
    poff_ref[...] = poff
    bnd = (poff + pc) / BM
    gi = lax.broadcasted_iota(jnp.int32, (NBCAP, E), 0).astype(_F32)
    eid = jnp.sum((gi >= bnd).astype(_F32), axis=-1, keepdims=True)
    eid_ref[...] = jnp.minimum(eid, E - 1).astype(jnp.int32)
    nreal_ref[...] = (jnp.sum(pc, axis=-1, keepdims=True) / BM).astype(jnp.int32)


def _meta(cnt):
    return pl.pallas_call(
        _meta_body,
        grid=(1,),
        in_specs=[pl.BlockSpec((1, E), lambda i: (0, 0))],
        out_specs=[
            pl.BlockSpec((1, E), lambda i: (0, 0)),
            pl.BlockSpec((NBCAP, 1), lambda i: (0, 0)),
            pl.BlockSpec((1, 1), lambda i: (0, 0)),
        ],
        out_shape=[
            jax.ShapeDtypeStruct((1, E), _F32),
            jax.ShapeDtypeStruct((NBCAP, 1), jnp.int32),
            jax.ShapeDtypeStruct((1, 1), jnp.int32),
        ],
    )(cnt)


def _pos_body(idx_ref, cum_ref, poff_ref, pos_ref):
    lane = lax.broadcasted_iota(jnp.int32, (BR, E), 1)
    base = cum_ref[...] + poff_ref[...]
    i0 = idx_ref[:, 0:1]
    i1 = idx_ref[:, 1:2]
    p0 = jnp.sum(jnp.where(lane == i0, base, 0.0), axis=-1, keepdims=True)
    p1 = jnp.sum(jnp.where(lane == i1, base, 0.0), axis=-1, keepdims=True)
    pos_ref[...] = jnp.concatenate([p0, p1], axis=-1).astype(jnp.int32)


def _pos(idx, cum, poff):
    return pl.pallas_call(
        _pos_body,
        grid=(T // BR,),
        in_specs=[
            pl.BlockSpec((BR, TOPK), lambda i: (i, 0)),
            pl.BlockSpec((BR, E), lambda i: (i, 0)),
            pl.BlockSpec((1, E), lambda i: (0, 0)),
        ],
        out_specs=pl.BlockSpec((BR, TOPK), lambda i: (i, 0)),
        out_shape=jax.ShapeDtypeStruct((T, TOPK), jnp.int32),
    )(idx, cum, poff)


def _gmm_body(eid_ref, nreal_ref, hg_ref, wg_ref, wu_ref, wd_ref, o_ref):
    g = pl.program_id(0)

    @pl.when(g < nreal_ref[0, 0])
    def _():
        a = hg_ref[...]
        gg = jnp.dot(a, wg_ref[0], preferred_element_type=_F32)
        uu = jnp.dot(a, wu_ref[0], preferred_element_type=_F32)
        act = gg * jax.nn.sigmoid(gg) * uu
        o_ref[...] = jnp.dot(act, wd_ref[0], preferred_element_type=_F32)


def _gmm(eid, nreal, hg, wg, wu, wd):
    grid_spec = pltpu.PrefetchScalarGridSpec(
        num_scalar_prefetch=2,
        grid=(NBCAP,),
        in_specs=[
            pl.BlockSpec((BM, D), lambda g, eid, nr: (g, 0)),
            pl.BlockSpec((1, D, I), lambda g, eid, nr: (eid[g, 0], 0, 0)),
            pl.BlockSpec((1, D, I), lambda g, eid, nr: (eid[g, 0], 0, 0)),
            pl.BlockSpec((1, I, D), lambda g, eid, nr: (eid[g, 0], 0, 0)),
        ],
        out_specs=pl.BlockSpec((BM, D), lambda g, eid, nr: (g, 0)),
    )
    return pl.pallas_call(
        _gmm_body,
        grid_spec=grid_spec,
        out_shape=jax.ShapeDtypeStruct((CAP, D), _F32),
    )(eid, nreal, hg, wg, wu, wd)


_NW = 32
_PW = (TOPK * T) // _NW
_NCH = _PW // 16
_TW = T // _NW


def _sc_mesh():
    return plsc.VectorSubcoreMesh(core_axis_name="c", subcore_axis_name="s")


def _moe_dispatch(h2, pos3, tok3):
    @functools.partial(
        pl.kernel,
        out_type=jax.ShapeDtypeStruct((CAP, D), _F32),
        mesh=_sc_mesh(),
        scratch_types=[
            pltpu.VMEM((_NCH, 16), jnp.int32),
            pltpu.VMEM((_NCH, 16), jnp.int32),
            pltpu.VMEM((16, D), _F32),
            pltpu.VMEM((16, D), _F32),
            pltpu.SemaphoreType.DMA,
            pltpu.SemaphoreType.DMA,
            pltpu.SemaphoreType.DMA,
            pltpu.SemaphoreType.DMA,
        ],
    )
    def k(h2_hbm, pos_hbm, tok_hbm, out_hbm, tokv, posv, bufa, bufb,
          ga, gb, sa, sb):
        wid = lax.axis_index("s") * 2 + lax.axis_index("c")
        pltpu.sync_copy(tok_hbm.at[wid], tokv)
        pltpu.sync_copy(pos_hbm.at[wid], posv)
        bufs = (bufa, bufb)
        gsem = (ga, gb)
        ssem = (sa, sb)
        gathers = [None] * _NCH
        scat = [None, None]
        gathers[0] = pltpu.async_copy(h2_hbm.at[tokv.at[0]], bufs[0], gsem[0])
        for j in range(_NCH):
            b = j % 2
            gathers[j].wait()
            if j + 1 < _NCH:
                nb = (j + 1) % 2
                if scat[nb] is not None:
                    scat[nb].wait()
                gathers[j + 1] = pltpu.async_copy(
                    h2_hbm.at[tokv.at[j + 1]], bufs[nb], gsem[nb])
            scat[b] = pltpu.async_copy(bufs[b], out_hbm.at[posv.at[j]],
                                       ssem[b])
        for b in range(2):
            if scat[b] is not None:
                scat[b].wait()

    return k(h2, pos3, tok3)


def _moe_combine(ys, pos3, wts3):
    @functools.partial(
        pl.kernel,
        out_type=jax.ShapeDtypeStruct((T, D), _F32),
        mesh=_sc_mesh(),
        scratch_types=[
            pltpu.VMEM((_NCH, 16), jnp.int32),
            pltpu.VMEM((_NCH, 16), _F32),
            pltpu.VMEM((16, D), _F32),
            pltpu.VMEM((8, D), _F32),
            pltpu.SemaphoreType.DMA,
            pltpu.SemaphoreType.DMA,
        ],
    )
    def k(ys_hbm, pos_hbm, wts_hbm, out_hbm, posv, wv, ybuf, obuf, gs, ss):
        wid = lax.axis_index("s") * 2 + lax.axis_index("c")
        pltpu.sync_copy(pos_hbm.at[wid], posv)
        pltpu.sync_copy(wts_hbm.at[wid], wv)
        base = wid * _TW
        for j in range(_NCH):
            pltpu.async_copy(ys_hbm.at[posv.at[j]], ybuf, gs).wait()
            wrow = wv[j]
            for i in range(8):
                w0 = wrow[2 * i]
                w1 = wrow[2 * i + 1]

                def body(c, carry, i=i, w0=w0, w1=w1):
                    sl = pl.ds(c * 16, 16)
                    obuf[i, sl] = w0 * ybuf[2 * i, sl] + w1 * ybuf[2 * i + 1, sl]
                    return carry

                lax.fori_loop(0, D // 16, body, 0)
            pltpu.sync_copy(obuf, out_hbm.at[pl.ds(base + 8 * j, 8)])

    return k(ys, pos3, wts3)


def kernel(positions, hidden_states, in_ln_w, post_ln_w, q_norm_w, k_norm_w,
           Wq, Wk, Wv, Wo, Wgate, Wg, Wu, Wd):
    wqkv = jnp.concatenate([Wq, Wk, Wv], axis=1)
    half = DH // 2
    inv_freq = 1.0 / (BASE ** (jnp.arange(0, half, dtype=_F32) / half))
    freqs = positions.astype(_F32)[:, None] * inv_freq[None, :]
    cos = jnp.cos(freqs)
    sin = jnp.sin(freqs)
    qkn = jnp.stack([q_norm_w, k_norm_w], axis=0).reshape(2, 1, DH)

    qkv = _rms_qkv(hidden_states, in_ln_w.reshape(1, D), wqkv)
    qk = _rope_qk(qkv[:, : (H + KV) * DH], qkn, cos, sin)
    attn = _flash(qk, qkv)
    new_res, h2, logits = _oproj(attn, Wo, hidden_states,
                                 post_ln_w.reshape(1, D), Wgate)
    idx, wts, cum, cnt = _route(logits)
    poff, eid, nreal = _meta(cnt)
    pos = _pos(idx, cum, poff)

    tok3 = jax.lax.broadcasted_iota(
        jnp.int32, (T, TOPK), 0).reshape(_NW, _NCH, 16)
    pos3 = pos.reshape(_NW, _NCH, 16)
    wts3 = wts.reshape(_NW, _NCH, 16)
    hg = _moe_dispatch(h2, pos3, tok3)
    ys = _gmm(eid, nreal, hg, Wg, Wu, Wd)
    moe_out = _moe_combine(ys, pos3, wts3)
    return (moe_out, new_res)
